# baseline (speedup 1.0000x reference)
import functools

import jax
import jax.numpy as jnp
import numpy as np
from jax import lax
from jax.experimental import pallas as pl
from jax.experimental.pallas import tpu as pltpu
from jax.experimental.pallas import tpu_sc as plsc


def _mm_body(x_ref, w_ref, o_ref):
    o_ref[...] = jnp.dot(x_ref[...], w_ref[...],
                         preferred_element_type=jnp.float32
                         ).astype(jnp.bfloat16)


def _matmul(x, w):
    n, d_in = x.shape
    d_out = w.shape[1]
    blk = 1000
    return pl.pallas_call(
        _mm_body,
        grid=(n // blk,),
        in_specs=[pl.BlockSpec((blk, d_in), lambda i: (i, 0)),
                  pl.BlockSpec((d_in, d_out), lambda i: (0, 0))],
        out_specs=pl.BlockSpec((blk, d_out), lambda i: (i, 0)),
        out_shape=jax.ShapeDtypeStruct((n, d_out), jnp.bfloat16),
    )(x, w)


def _comb_body(p_ref, b_ref, o_ref):
    o_ref[...] = p_ref[0] + p_ref[1] + b_ref[...]


def _combine(p, b2d):
    nc, n, d = p.shape
    blk = 1000
    return pl.pallas_call(
        _comb_body,
        grid=(n // blk,),
        in_specs=[pl.BlockSpec((nc, blk, d), lambda i: (0, i, 0)),
                  pl.BlockSpec((1, d), lambda i: (0, 0))],
        out_specs=pl.BlockSpec((blk, d), lambda i: (i, 0)),
        out_shape=jax.ShapeDtypeStruct((n, d), jnp.float32),
    )(p, b2d)


def _qperm(d):
    q = np.empty((d,), dtype=np.int32)
    for g in range(d // 32):
        for i in range(16):
            q[32 * g + 2 * i] = 32 * g + i
            q[32 * g + 2 * i + 1] = 32 * g + 16 + i
    return q


@functools.lru_cache(maxsize=None)
def _make_sc_spmm(n, d, e):
    info = plsc.get_sparse_core_info()
    nc, ns, nl = info.num_cores, info.num_subcores, info.num_lanes
    nw = nc * ns
    d2 = d // 2
    epw = e // nw
    chunk_e = 64
    blk_chunks = 16
    n_blocks = -(-epw // (chunk_e * blk_chunks))
    epw_pad = n_blocks * blk_chunks * chunk_e
    n_chunks = epw_pad // chunk_e
    sl_rows = 400
    n_slices = n // sl_rows
    sl_per_tile = -(-n_slices // ns)
    zrows = 80
    mesh = plsc.VectorSubcoreMesh(core_axis_name="c", subcore_axis_name="s")

    @functools.partial(
        pl.kernel,
        out_type=jax.ShapeDtypeStruct((nc, n, d), jnp.float32),
        mesh=mesh,
        compiler_params=pltpu.CompilerParams(needs_layout_passes=False, use_tc_tiling_on_sc=False),
        scratch_types=[
            pltpu.VMEM((blk_chunks, chunk_e), jnp.int32),
            pltpu.VMEM((blk_chunks, chunk_e), jnp.int32),
            pltpu.VMEM((blk_chunks, chunk_e), jnp.float32),
            pltpu.VMEM((chunk_e, d2), jnp.uint32),
            pltpu.VMEM((chunk_e, d2), jnp.uint32),
            pltpu.VMEM((chunk_e, d2), jnp.uint32),
            pltpu.VMEM((chunk_e, d2), jnp.uint32),
            pltpu.VMEM((chunk_e, d), jnp.float32),
            pltpu.VMEM_SHARED((n, d), jnp.float32),
            pltpu.SemaphoreType.DMA,
            pltpu.SemaphoreType.DMA,
            pltpu.SemaphoreType.DMA,
            pltpu.SemaphoreType.DMA,
        ],
    )
    def spmm(support_hbm, rows3_hbm, cols3_hbm, vals3_hbm, out_hbm,
             cidx_v, ridx_v, vals_v, gat_0, gat_1, gat_2, gat_3, sbuf,
             acc_sh, gsem_0, gsem_1, gsem_2, gsem_3):
        cid = lax.axis_index("c")
        sid = lax.axis_index("s")
        wid = sid * nc + cid

        zero16 = jnp.zeros((nl,), jnp.float32)

        def zrow(i, carry):
            for j in range(d // nl):
                sbuf[i, pl.ds(j * nl, nl)] = zero16
            return carry
        lax.fori_loop(0, zrows, zrow, 0)
        zsrc = sbuf.at[pl.ds(0, zrows)]
        for k in range(sl_per_tile):
            sl_id = sid + ns * k
            @pl.when(sl_id < n_slices)
            def _():
                off = pl.multiple_of(sl_id * sl_rows, 8)
                for z in range(sl_rows // zrows):
                    pltpu.sync_copy(zsrc,
                                    acc_sh.at[pl.ds(off + z * zrows, zrows)])
        plsc.subcore_barrier()

        def start_gather(c, buf, sem):
            pltpu.async_copy(support_hbm.at[cidx_v.at[c]], buf, sem)

        def wait_gather(c, buf, sem):
            pltpu.make_async_copy(support_hbm.at[cidx_v.at[c]], buf,
                                  sem).wait()

        hi_mask = jnp.full((nl,), 0xFFFF0000, jnp.uint32)
        shift16 = jnp.full((nl,), 16, jnp.uint32)

        def scale(c, buf):
            def grp(g, c2):
                vblock = vals_v[c, pl.ds(g * nl, nl)]
                base_e = g * nl
                for k in range(nl):
                    v = vblock[k]
                    ei = base_e + k
                    for j in range(d2 // nl):
                        u = buf[ei, pl.ds(nl * j, nl)]
                        lo = plsc.bitcast(u << shift16, jnp.float32)
                        hi = plsc.bitcast(u & hi_mask, jnp.float32)
                        sbuf[ei, pl.ds(2 * nl * j, nl)] = lo * v
                        sbuf[ei, pl.ds(2 * nl * j + nl, nl)] = hi * v
                return c2
            lax.fori_loop(0, chunk_e // nl, grp, 0)

        def scatter(c):
            pltpu.sync_copy(sbuf, acc_sh.at[ridx_v.at[c]], add=True)

        bufs = (gat_0, gat_1, gat_2, gat_3)
        sems = (gsem_0, gsem_1, gsem_2, gsem_3)
        nbuf = 4

        def block(b, carry):
            pltpu.sync_copy(cols3_hbm.at[wid, b], cidx_v)
            pltpu.sync_copy(rows3_hbm.at[wid, b], ridx_v)
            pltpu.sync_copy(vals3_hbm.at[wid, b], vals_v)

            for k in range(nbuf):
                start_gather(k, bufs[k], sems[k])

            def quad(i, c2):
                c0 = nbuf * i
                for k in range(nbuf):
                    c = c0 + k
                    wait_gather(c, bufs[k], sems[k])
                    scale(c, bufs[k])
                    scatter(c)
                    @pl.when(c + nbuf < blk_chunks)
                    def _():
                        start_gather(c + nbuf, bufs[k], sems[k])
                return c2
            lax.fori_loop(0, blk_chunks // nbuf, quad, 0)
            return carry
        lax.fori_loop(0, n_blocks, block, 0)

        plsc.subcore_barrier()
        for k in range(sl_per_tile):
            sl_id = sid + ns * k
            @pl.when(sl_id < n_slices)
            def _():
                off = pl.multiple_of(sl_id * sl_rows, 8)
                pltpu.sync_copy(acc_sh.at[pl.ds(off, sl_rows)],
                                out_hbm.at[cid, pl.ds(off, sl_rows)])

    return spmm, nw, n_blocks, blk_chunks, chunk_e, epw, epw_pad


def kernel(node_features, adj_indices, adj_values, W, b):
    n, _ = node_features.shape
    d = W.shape[1]
    e = adj_values.shape[0]
    support = _matmul(node_features, W[:, _qperm(d)])
    support_u32 = lax.bitcast_convert_type(
        support.reshape(n, d // 2, 2), jnp.uint32)
    spmm, nw, n_blocks, blk_chunks, chunk_e, epw, epw_pad = _make_sc_spmm(
        n, d, e)
    pad = ((0, 0), (0, epw_pad - epw))
    shp = (nw, n_blocks, blk_chunks, chunk_e)
    rows3 = jnp.pad(adj_indices[0].reshape(nw, epw), pad).reshape(shp)
    cols3 = jnp.pad(adj_indices[1].reshape(nw, epw), pad).reshape(shp)
    vals3 = jnp.pad(adj_values.reshape(nw, epw), pad).reshape(shp)
    partials = spmm(support_u32, rows3, cols3, vals3)
    return _combine(partials, b.reshape(1, d))


# speedup vs baseline: 1.2409x; 1.2409x over previous
import functools

import jax
import jax.numpy as jnp
import numpy as np
from jax import lax
from jax.experimental import pallas as pl
from jax.experimental.pallas import tpu as pltpu
from jax.experimental.pallas import tpu_sc as plsc


def _mm_body(x_ref, w_ref, o_ref):
    o_ref[...] = jnp.dot(x_ref[...], w_ref[...],
                         preferred_element_type=jnp.float32
                         ).astype(jnp.bfloat16)


def _matmul(x, w):
    n, d_in = x.shape
    d_out = w.shape[1]
    blk = 1000
    return pl.pallas_call(
        _mm_body,
        grid=(n // blk,),
        in_specs=[pl.BlockSpec((blk, d_in), lambda i: (i, 0)),
                  pl.BlockSpec((d_in, d_out), lambda i: (0, 0))],
        out_specs=pl.BlockSpec((blk, d_out), lambda i: (i, 0)),
        out_shape=jax.ShapeDtypeStruct((n, d_out), jnp.bfloat16),
    )(x, w)


def _comb_body(p_ref, b_ref, o_ref):
    o_ref[...] = p_ref[0] + p_ref[1] + b_ref[...]


def _combine(p, b2d):
    nc, n, d = p.shape
    blk = 1000
    return pl.pallas_call(
        _comb_body,
        grid=(n // blk,),
        in_specs=[pl.BlockSpec((nc, blk, d), lambda i: (0, i, 0)),
                  pl.BlockSpec((1, d), lambda i: (0, 0))],
        out_specs=pl.BlockSpec((blk, d), lambda i: (i, 0)),
        out_shape=jax.ShapeDtypeStruct((n, d), jnp.float32),
    )(p, b2d)


def _qperm(d):
    q = np.empty((d,), dtype=np.int32)
    for g in range(d // 32):
        for i in range(16):
            q[32 * g + 2 * i] = 32 * g + i
            q[32 * g + 2 * i + 1] = 32 * g + 16 + i
    return q


@functools.lru_cache(maxsize=None)
def _make_sc_spmm(n, d, e):
    info = plsc.get_sparse_core_info()
    nc, ns, nl = info.num_cores, info.num_subcores, info.num_lanes
    nw = nc * ns
    d2 = d // 2
    epw = e // nw
    chunk_e = 112
    blk_chunks = 10
    n_blocks = -(-epw // (chunk_e * blk_chunks))
    epw_pad = n_blocks * blk_chunks * chunk_e
    n_chunks = epw_pad // chunk_e
    sl_rows = 400
    n_slices = n // sl_rows
    sl_per_tile = -(-n_slices // ns)
    zrows = 80
    mesh = plsc.VectorSubcoreMesh(core_axis_name="c", subcore_axis_name="s")

    @functools.partial(
        pl.kernel,
        out_type=jax.ShapeDtypeStruct((nc, n, d), jnp.float32),
        mesh=mesh,
        compiler_params=pltpu.CompilerParams(needs_layout_passes=False, use_tc_tiling_on_sc=False),
        scratch_types=[
            pltpu.VMEM((blk_chunks, chunk_e), jnp.int32),
            pltpu.VMEM((blk_chunks, chunk_e), jnp.int32),
            pltpu.VMEM((blk_chunks, chunk_e), jnp.float32),
            pltpu.VMEM((chunk_e, d2), jnp.uint32),
            pltpu.VMEM((chunk_e, d2), jnp.uint32),
            pltpu.VMEM((chunk_e, d), jnp.float32),
            pltpu.VMEM_SHARED((n, d), jnp.float32),
            pltpu.SemaphoreType.DMA,
            pltpu.SemaphoreType.DMA,
        ],
    )
    def spmm(support_hbm, rows3_hbm, cols3_hbm, vals3_hbm, out_hbm,
             cidx_v, ridx_v, vals_v, gat_a, gat_b, sbuf, acc_sh,
             gsem_a, gsem_b):
        cid = lax.axis_index("c")
        sid = lax.axis_index("s")
        wid = sid * nc + cid

        zero16 = jnp.zeros((nl,), jnp.float32)

        def zrow(i, carry):
            for j in range(d // nl):
                sbuf[i, pl.ds(j * nl, nl)] = zero16
            return carry
        lax.fori_loop(0, zrows, zrow, 0)
        zsrc = sbuf.at[pl.ds(0, zrows)]
        for k in range(sl_per_tile):
            sl_id = sid + ns * k
            @pl.when(sl_id < n_slices)
            def _():
                off = pl.multiple_of(sl_id * sl_rows, 8)
                for z in range(sl_rows // zrows):
                    pltpu.sync_copy(zsrc,
                                    acc_sh.at[pl.ds(off + z * zrows, zrows)])
        plsc.subcore_barrier()

        def start_gather(c, buf, sem):
            pltpu.async_copy(support_hbm.at[cidx_v.at[c]], buf, sem)

        def wait_gather(c, buf, sem):
            pltpu.make_async_copy(support_hbm.at[cidx_v.at[c]], buf,
                                  sem).wait()

        hi_mask = jnp.full((nl,), 0xFFFF0000, jnp.uint32)
        shift16 = jnp.full((nl,), 16, jnp.uint32)

        def scale(c, buf):
            def grp(g, c2):
                vblock = vals_v[c, pl.ds(g * nl, nl)]
                base_e = g * nl
                for k in range(nl):
                    v = vblock[k]
                    ei = base_e + k
                    for j in range(d2 // nl):
                        u = buf[ei, pl.ds(nl * j, nl)]
                        lo = plsc.bitcast(u << shift16, jnp.float32)
                        hi = plsc.bitcast(u & hi_mask, jnp.float32)
                        sbuf[ei, pl.ds(2 * nl * j, nl)] = lo * v
                        sbuf[ei, pl.ds(2 * nl * j + nl, nl)] = hi * v
                return c2
            lax.fori_loop(0, chunk_e // nl, grp, 0)

        def scatter(c):
            pltpu.sync_copy(sbuf, acc_sh.at[ridx_v.at[c]], add=True)

        def block(b, carry):
            pltpu.sync_copy(cols3_hbm.at[wid, b], cidx_v)
            pltpu.sync_copy(rows3_hbm.at[wid, b], ridx_v)
            pltpu.sync_copy(vals3_hbm.at[wid, b], vals_v)

            start_gather(0, gat_a, gsem_a)
            start_gather(1, gat_b, gsem_b)

            def pair(i, c2):
                c0 = 2 * i
                wait_gather(c0, gat_a, gsem_a)
                scale(c0, gat_a)
                scatter(c0)
                @pl.when(c0 + 2 < blk_chunks)
                def _():
                    start_gather(c0 + 2, gat_a, gsem_a)
                wait_gather(c0 + 1, gat_b, gsem_b)
                scale(c0 + 1, gat_b)
                scatter(c0 + 1)
                @pl.when(c0 + 3 < blk_chunks)
                def _():
                    start_gather(c0 + 3, gat_b, gsem_b)
                return c2
            lax.fori_loop(0, blk_chunks // 2, pair, 0)
            return carry
        lax.fori_loop(0, n_blocks, block, 0)

        plsc.subcore_barrier()
        for k in range(sl_per_tile):
            sl_id = sid + ns * k
            @pl.when(sl_id < n_slices)
            def _():
                off = pl.multiple_of(sl_id * sl_rows, 8)
                pltpu.sync_copy(acc_sh.at[pl.ds(off, sl_rows)],
                                out_hbm.at[cid, pl.ds(off, sl_rows)])

    return spmm, nw, n_blocks, blk_chunks, chunk_e, epw, epw_pad


def kernel(node_features, adj_indices, adj_values, W, b):
    n, _ = node_features.shape
    d = W.shape[1]
    e = adj_values.shape[0]
    support = _matmul(node_features, W[:, _qperm(d)])
    support_u32 = lax.bitcast_convert_type(
        support.reshape(n, d // 2, 2), jnp.uint32)
    spmm, nw, n_blocks, blk_chunks, chunk_e, epw, epw_pad = _make_sc_spmm(
        n, d, e)
    pad = ((0, 0), (0, epw_pad - epw))
    shp = (nw, n_blocks, blk_chunks, chunk_e)
    rows3 = jnp.pad(adj_indices[0].reshape(nw, epw), pad).reshape(shp)
    cols3 = jnp.pad(adj_indices[1].reshape(nw, epw), pad).reshape(shp)
    vals3 = jnp.pad(adj_values.reshape(nw, epw), pad).reshape(shp)
    partials = spmm(support_u32, rows3, cols3, vals3)
    return _combine(partials, b.reshape(1, d))
